# baseline (speedup 1.0000x reference)
import functools

import jax
import jax.numpy as jnp
from jax import lax
from jax.experimental import pallas as pl
from jax.experimental.pallas import tpu as pltpu
from jax.experimental.pallas import tpu_sc as plsc

EMBED_DIM = 64
NC = 2
NS = 16
NW = NC * NS
EPB = 2
NGRP = 3


def _emb_body(table, idx_hbm, out, idx_v, rows_v, gs0, gs1, gs2, os0, os1, os2):
    wid = lax.axis_index("s") * NC + lax.axis_index("c")
    e_per_w = idx_hbm.shape[0] // NW
    seq = idx_hbm.shape[1]
    nb = e_per_w // EPB
    base = wid * e_per_w

    pltpu.sync_copy(idx_hbm.at[pl.ds(base, e_per_w)], idx_v)

    gsems = (gs0, gs1, gs2)
    osems = (os0, os1, os2)
    splits = [(0, 128), (128, seq - 128)]

    def fire_g(h, grp):
        for i in range(EPB):
            for (o, n) in splits:
                pltpu.async_copy(table.at[idx_v.at[h * EPB + i, pl.ds(o, n)]],
                                 rows_v.at[grp, i, pl.ds(o, n)], gsems[grp])

    def wait_g(h, grp):
        for i in range(EPB):
            for (o, n) in splits:
                pltpu.make_async_copy(table.at[idx_v.at[h * EPB + i, pl.ds(o, n)]],
                                      rows_v.at[grp, i, pl.ds(o, n)],
                                      gsems[grp]).wait()

    def fire_s(h, grp):
        pltpu.async_copy(rows_v.at[grp], out.at[pl.ds(base + h * EPB, EPB)],
                         osems[grp])

    def wait_s(h, grp):
        pltpu.make_async_copy(rows_v.at[grp], out.at[pl.ds(base + h * EPB, EPB)],
                              osems[grp]).wait()

    def step(h, grp, fire=True):
        wait_s(h - 1, (grp + 2) % NGRP)
        if fire:
            fire_g(h + 2, (grp + 2) % NGRP)
        wait_g(h, grp)
        fire_s(h, grp)

    fire_g(0, 0)
    fire_g(1, 1)
    fire_g(2, 2)
    wait_g(0, 0)
    fire_s(0, 0)
    step(1, 1)
    step(2, 2)

    def outer(t, c):
        h0 = t * NGRP
        for dh in range(NGRP):
            step(h0 + dh, dh)
        return c

    t_end = (nb - 2) // NGRP
    lax.fori_loop(1, t_end, outer, 0)

    for h in range(NGRP * t_end, nb):
        step(h, h % NGRP, fire=(h + 2 < nb))
    wait_s(nb - 1, (nb - 1) % NGRP)


def _make_emb(bs, seq):
    e_per_w = bs // NW
    return functools.partial(
        pl.kernel,
        out_type=jax.ShapeDtypeStruct((bs, seq, EMBED_DIM), jnp.float32),
        mesh=plsc.VectorSubcoreMesh(core_axis_name="c", subcore_axis_name="s"),
        scratch_types=[
            pltpu.VMEM((e_per_w, seq), jnp.int32),
            pltpu.VMEM((NGRP, EPB, seq, EMBED_DIM), jnp.float32),
        ] + [pltpu.SemaphoreType.DMA] * (2 * NGRP),
        compiler_params=pltpu.CompilerParams(use_tc_tiling_on_sc=False),
    )(_emb_body)


def kernel(token_ids, weight):
    bs, seq = token_ids.shape
    return _make_emb(bs, seq)(weight, token_ids)


# speedup vs baseline: 1.3346x; 1.3346x over previous
import functools

import jax
import jax.numpy as jnp
from jax import lax
from jax.experimental import pallas as pl
from jax.experimental.pallas import tpu as pltpu
from jax.experimental.pallas import tpu_sc as plsc

EMBED_DIM = 64
PD = 2 * EMBED_DIM
NC = 2
NS = 16
NW = NC * NS
NGRP = 3


def _emb_body(table, idx_hbm, out, idx_v, rows_v, gs0, gs1, gs2, os0, os1, os2):
    wid = lax.axis_index("s") * NC + lax.axis_index("c")
    e_per_w = idx_hbm.shape[0] // NW
    seq = idx_hbm.shape[1]
    base = wid * e_per_w

    pltpu.sync_copy(idx_hbm.at[pl.ds(base, e_per_w)], idx_v)

    gsems = (gs0, gs1, gs2)
    osems = (os0, os1, os2)
    splits = [(0, 128), (128, seq - 128)]

    def fire_g(b, grp):
        for (o, n) in splits:
            pltpu.async_copy(table.at[idx_v.at[b, pl.ds(o, n)]],
                             rows_v.at[grp, pl.ds(o, n)], gsems[grp])

    def wait_g(b, grp):
        for (o, n) in splits:
            pltpu.make_async_copy(table.at[idx_v.at[b, pl.ds(o, n)]],
                                  rows_v.at[grp, pl.ds(o, n)],
                                  gsems[grp]).wait()

    def fire_s(b, grp):
        pltpu.async_copy(rows_v.at[grp],
                         out.at[base + b, pl.ds(0, seq), pl.ds(0, EMBED_DIM)],
                         osems[grp])

    def wait_s(b, grp):
        pltpu.make_async_copy(rows_v.at[grp],
                              out.at[base + b, pl.ds(0, seq), pl.ds(0, EMBED_DIM)],
                              osems[grp]).wait()

    def step(h, grp, fire=True):
        wait_s(h - 1, (grp + 2) % NGRP)
        if fire:
            fire_g(h + 2, (grp + 2) % NGRP)
        wait_g(h, grp)
        fire_s(h, grp)

    fire_g(0, 0)
    fire_g(1, 1)
    fire_g(2, 2)
    wait_g(0, 0)
    fire_s(0, 0)
    step(1, 1)
    step(2, 2)

    def outer(t, c):
        h0 = t * NGRP
        for dh in range(NGRP):
            step(h0 + dh, dh)
        return c

    t_end = (e_per_w - 2) // NGRP
    lax.fori_loop(1, t_end, outer, 0)
    for h in range(NGRP * t_end, e_per_w):
        step(h, h % NGRP, fire=(h + 2 < e_per_w))
    wait_s(e_per_w - 1, (e_per_w - 1) % NGRP)


def _make_emb(bs, seq):
    return functools.partial(
        pl.kernel,
        out_type=jax.ShapeDtypeStruct((bs, seq, PD), jnp.float32),
        mesh=plsc.VectorSubcoreMesh(core_axis_name="c", subcore_axis_name="s"),
        scratch_types=[
            pltpu.VMEM((bs // NW, seq), jnp.int32),
            pltpu.VMEM((NGRP, seq, EMBED_DIM), jnp.float32),
        ] + [pltpu.SemaphoreType.DMA] * (2 * NGRP),
        compiler_params=pltpu.CompilerParams(use_tc_tiling_on_sc=False),
    )(_emb_body)


def kernel(token_ids, weight):
    bs, seq = token_ids.shape
    padded = _make_emb(bs, seq)(weight, token_ids)
    return padded[:, :, :EMBED_DIM]
